# baseline (speedup 1.0000x reference)
import jax
import jax.numpy as jnp
from jax import lax
from jax.experimental import pallas as pl
from jax.experimental.pallas import tpu as pltpu
from jax.experimental.pallas import tpu_sc as plsc

N_NODES = 10000
N_EDGES = 320000
D_FEAT = 128
W_ROW = D_FEAT // 2

NC = 2
NS = 16
NW = NC * NS
LANES = 16

EPW = N_EDGES // NW
C = 80
NCH = EPW // C
GROUPS = C // LANES


def _tec_body(h_hbm, comb_hbm, b_hbm, out_hbm,
              b_v, icomb, out_v, ra, rb, h_sh, sem_a, sem_b):
  cid = lax.axis_index("c")
  sid = lax.axis_index("s")
  wid = sid * NC + cid
  base = wid * EPW

  slab = N_NODES // NS
  pltpu.sync_copy(h_hbm.at[pl.ds(sid * slab, slab)],
                  h_sh.at[pl.ds(sid * slab, slab)])

  pltpu.sync_copy(b_hbm, b_v)
  pltpu.sync_copy(comb_hbm.at[wid], icomb)
  plsc.subcore_barrier()

  lane = lax.iota(jnp.int32, LANES)

  def start(j, rbuf, sem):
    pltpu.async_copy(h_sh.at[icomb.at[j]], rbuf, sem)

  def wait(j, rbuf, sem):
    pltpu.make_async_copy(h_sh.at[icomb.at[j]], rbuf, sem).wait()

  def compute(j, rbuf):
    @plsc.parallel_loop(0, GROUPS, unroll=2)
    def group(g):
      e0 = j * C + g * LANES
      row16 = g * LANES + lane
      drow16 = C + g * LANES + lane
      acc_a = jnp.zeros((LANES,), jnp.float32)
      acc_b = jnp.zeros((LANES,), jnp.float32)
      for f in range(W_ROW):
        col = lane ^ f
        sv = plsc.bitcast(plsc.load_gather(rbuf, [row16, col]), jnp.bfloat16)
        dv = plsc.bitcast(plsc.load_gather(rbuf, [drow16, col]), jnp.bfloat16)
        pa, pb = plsc.unpack(sv * dv, format=plsc.PackFormat.INTERLEAVED)
        acc_a = acc_a + pa
        acc_b = acc_b + pb
      si = icomb[j, pl.ds(g * LANES, LANES)]
      di = icomb[j, pl.ds(C + g * LANES, LANES)]
      acc = (acc_a + acc_b) + (
          plsc.load_gather(b_v, [si]) + plsc.load_gather(b_v, [di]))
      out_v[pl.ds(e0, LANES)] = acc
    del group

  start(0, ra, sem_a)

  def body(t, _):
    j = 2 * t
    start(j + 1, rb, sem_b)
    wait(j, ra, sem_a)
    compute(j, ra)
    start(j + 2, ra, sem_a)
    wait(j + 1, rb, sem_b)
    compute(j + 1, rb)
    return 0

  lax.fori_loop(0, NCH // 2, body, 0)
  wait(NCH - 1, ra, sem_a)
  compute(NCH - 1, ra)

  pltpu.sync_copy(out_v, out_hbm.at[pl.ds(base, EPW)])


@jax.jit
def _run(h_packed, comb, b_flat):
  mesh = plsc.VectorSubcoreMesh(
      core_axis_name="c", subcore_axis_name="s", num_cores=NC, num_subcores=NS)
  fn = pl.kernel(
      _tec_body,
      out_type=jax.ShapeDtypeStruct((N_EDGES,), jnp.float32),
      mesh=mesh,
      scratch_types=[
          pltpu.VMEM((N_NODES,), jnp.float32),
          pltpu.VMEM((NCH, 2 * C), jnp.int32),
          pltpu.VMEM((EPW,), jnp.float32),
          pltpu.VMEM((2 * C, W_ROW), jnp.int32),
          pltpu.VMEM((2 * C, W_ROW), jnp.int32),
          pltpu.VMEM_SHARED((N_NODES, W_ROW), jnp.int32),
          pltpu.SemaphoreType.DMA,
          pltpu.SemaphoreType.DMA,
      ],
      compiler_params=pltpu.CompilerParams(
          needs_layout_passes=False, use_tc_tiling_on_sc=False),
  )
  return fn(h_packed, comb, b_flat)


def kernel(h, edge_index, b):
  src = edge_index[0].astype(jnp.int32)
  dst = edge_index[1].astype(jnp.int32)
  h_bf = h.astype(jnp.bfloat16).reshape(N_NODES, W_ROW, 2)
  h_packed = lax.bitcast_convert_type(h_bf, jnp.int32)
  comb = jnp.concatenate(
      [src.reshape(NW, NCH, C), dst.reshape(NW, NCH, C)], axis=2)
  out = _run(h_packed, comb, b[:, 0])
  return out.reshape(N_EDGES, 1)


# speedup vs baseline: 1.1085x; 1.1085x over previous
import jax
import jax.numpy as jnp
from jax import lax
from jax.experimental import pallas as pl
from jax.experimental.pallas import tpu as pltpu
from jax.experimental.pallas import tpu_sc as plsc

N_NODES = 10000
N_EDGES = 320000
D_FEAT = 128
W_ROW = D_FEAT // 2

NC = 2
NS = 16
NW = NC * NS
LANES = 16

EPW = N_EDGES // NW
C = 80
NCH = EPW // C
GROUPS = C // LANES


def _tec_body(h_hbm, src_hbm, dst_hbm, b_hbm, out_hbm,
              b_v, isa, ida, out_v, rsa, rda, rsb, rdb, h_sh,
              sem_sa, sem_da, sem_sb, sem_db):
  wid = lax.axis_index("s") * NC + lax.axis_index("c")
  base = wid * EPW

  sid = lax.axis_index("s")
  slab = N_NODES // NS
  pltpu.sync_copy(h_hbm.at[pl.ds(sid * slab, slab)],
                  h_sh.at[pl.ds(sid * slab, slab)])

  pltpu.sync_copy(b_hbm, b_v)
  pltpu.sync_copy(src_hbm.at[pl.ds(base, EPW)], isa)
  pltpu.sync_copy(dst_hbm.at[pl.ds(base, EPW)], ida)
  plsc.subcore_barrier()

  lane = lax.iota(jnp.int32, LANES)

  def start(j, rs, rd, sem_s, sem_d):
    pltpu.async_copy(h_sh.at[isa.at[pl.ds(j * C, C)]], rs, sem_s)
    pltpu.async_copy(h_sh.at[ida.at[pl.ds(j * C, C)]], rd, sem_d)

  def wait(j, rs, rd, sem_s, sem_d):
    pltpu.make_async_copy(h_sh.at[isa.at[pl.ds(j * C, C)]], rs, sem_s).wait()
    pltpu.make_async_copy(h_sh.at[ida.at[pl.ds(j * C, C)]], rd, sem_d).wait()

  def compute(j, rs, rd):
    @plsc.parallel_loop(0, GROUPS, unroll=2)
    def group(g):
      e0 = j * C + g * LANES
      row16 = g * LANES + lane
      acc_a = jnp.zeros((LANES,), jnp.float32)
      acc_b = jnp.zeros((LANES,), jnp.float32)
      for f in range(W_ROW):
        col = lane ^ f
        sv = plsc.bitcast(plsc.load_gather(rs, [row16, col]), jnp.bfloat16)
        dv = plsc.bitcast(plsc.load_gather(rd, [row16, col]), jnp.bfloat16)
        pa, pb = plsc.unpack(sv * dv, format=plsc.PackFormat.INTERLEAVED)
        acc_a = acc_a + pa
        acc_b = acc_b + pb
      si = isa[pl.ds(e0, LANES)]
      di = ida[pl.ds(e0, LANES)]
      acc = (acc_a + acc_b) + (
          plsc.load_gather(b_v, [si]) + plsc.load_gather(b_v, [di]))
      out_v[pl.ds(e0, LANES)] = acc
    del group

  start(0, rsa, rda, sem_sa, sem_da)

  def body(t, _):
    j = 2 * t
    start(j + 1, rsb, rdb, sem_sb, sem_db)
    wait(j, rsa, rda, sem_sa, sem_da)
    compute(j, rsa, rda)
    start(j + 2, rsa, rda, sem_sa, sem_da)
    wait(j + 1, rsb, rdb, sem_sb, sem_db)
    compute(j + 1, rsb, rdb)
    return 0

  lax.fori_loop(0, NCH // 2, body, 0)
  wait(NCH - 1, rsa, rda, sem_sa, sem_da)
  compute(NCH - 1, rsa, rda)

  pltpu.sync_copy(out_v, out_hbm.at[pl.ds(base, EPW)])


@jax.jit
def _run(h_packed, src, dst, b_flat):
  mesh = plsc.VectorSubcoreMesh(
      core_axis_name="c", subcore_axis_name="s", num_cores=NC, num_subcores=NS)
  fn = pl.kernel(
      _tec_body,
      out_type=jax.ShapeDtypeStruct((N_EDGES,), jnp.float32),
      mesh=mesh,
      scratch_types=[
          pltpu.VMEM((N_NODES,), jnp.float32),
          pltpu.VMEM((EPW,), jnp.int32),
          pltpu.VMEM((EPW,), jnp.int32),
          pltpu.VMEM((EPW,), jnp.float32),
          pltpu.VMEM((C, W_ROW), jnp.int32),
          pltpu.VMEM((C, W_ROW), jnp.int32),
          pltpu.VMEM((C, W_ROW), jnp.int32),
          pltpu.VMEM((C, W_ROW), jnp.int32),
          pltpu.VMEM_SHARED((N_NODES, W_ROW), jnp.int32),
          pltpu.SemaphoreType.DMA,
          pltpu.SemaphoreType.DMA,
          pltpu.SemaphoreType.DMA,
          pltpu.SemaphoreType.DMA,
      ],
      compiler_params=pltpu.CompilerParams(
          needs_layout_passes=False, use_tc_tiling_on_sc=False),
  )
  return fn(h_packed, src, dst, b_flat)


def kernel(h, edge_index, b):
  src = edge_index[0].astype(jnp.int32)
  dst = edge_index[1].astype(jnp.int32)
  h_bf = h.astype(jnp.bfloat16).reshape(N_NODES, W_ROW, 2)
  h_packed = lax.bitcast_convert_type(h_bf, jnp.int32)
  out = _run(h_packed, src, dst, b[:, 0])
  return out.reshape(N_EDGES, 1)
